# baseline (speedup 1.0000x reference)
import functools

import jax
import jax.numpy as jnp
from jax import lax
from jax.experimental import pallas as pl
from jax.experimental.pallas import tpu as pltpu
from jax.experimental.pallas import tpu_sc as plsc

N = 10000
D_IN = 128
D_H = 64
D_OUT = 32

NC = 2
NS = 16
NW = NC * NS
L = 16

NP = 10240
CH = 128
E = 320000
EC_T = 79
E_PAD = NW * EC_T * CH
E2 = 2 * E
DC_T = 157
E2_PAD = NW * DC_T * CH

@functools.cache
def _mesh():
  return plsc.VectorSubcoreMesh(core_axis_name="c", subcore_axis_name="s",
                                num_cores=NC, num_subcores=NS)


_RB = 1000


def _enc_body(x_ref, win_ref, bin_ref, wl_ref, wr_ref, h_ref, xl_ref, xr_ref):
  h = jnp.dot(x_ref[...], win_ref[...], preferred_element_type=jnp.float32)
  h = h + bin_ref[...]
  h_ref[...] = h
  xl_ref[...] = jnp.dot(h, wl_ref[...], preferred_element_type=jnp.float32)
  xr_ref[...] = jnp.dot(h, wr_ref[...], preferred_element_type=jnp.float32)


def _encode(x, W_in, b_in, W_l, W_r):
  grid = (N // _RB,)
  out = pl.pallas_call(
      _enc_body,
      grid=grid,
      in_specs=[
          pl.BlockSpec((_RB, D_IN), lambda i: (i, 0)),
          pl.BlockSpec((D_IN, D_H), lambda i: (0, 0)),
          pl.BlockSpec((1, D_H), lambda i: (0, 0)),
          pl.BlockSpec((D_H, D_H), lambda i: (0, 0)),
          pl.BlockSpec((D_H, D_H), lambda i: (0, 0)),
      ],
      out_specs=[
          pl.BlockSpec((_RB, D_H), lambda i: (i, 0)),
          pl.BlockSpec((_RB, D_H), lambda i: (i, 0)),
          pl.BlockSpec((_RB, D_H), lambda i: (i, 0)),
      ],
      out_shape=[
          jax.ShapeDtypeStruct((N, D_H), jnp.float32),
          jax.ShapeDtypeStruct((N, D_H), jnp.float32),
          jax.ShapeDtypeStruct((N, D_H), jnp.float32),
      ],
  )(x, W_in, b_in.reshape(1, D_H), W_l, W_r)
  return out


@functools.cache
def _sc_edge_kernel():
  return pl.kernel(
      _sc_edge,
      out_type=[
          jax.ShapeDtypeStruct((NC, NP, D_H), jnp.float32),
          jax.ShapeDtypeStruct((NC, NP), jnp.float32),
      ],
      mesh=_mesh(),
      compiler_params=pltpu.CompilerParams(needs_layout_passes=False, use_tc_tiling_on_sc=False),
      scratch_types=[
          pltpu.VMEM((CH,), jnp.int32),
          pltpu.VMEM((CH,), jnp.int32),
          pltpu.VMEM((CH, D_H), jnp.float32),
          pltpu.VMEM((CH, D_H), jnp.float32),
          pltpu.VMEM((CH, D_H), jnp.float32),
          pltpu.VMEM((CH,), jnp.float32),
          pltpu.VMEM((D_H,), jnp.float32),
          pltpu.VMEM_SHARED((NP, D_H), jnp.float32),
          pltpu.VMEM_SHARED((NP,), jnp.float32),
          pltpu.SemaphoreType.DMA,
          pltpu.SemaphoreType.DMA,
      ],
  )


def _sc_edge(xl_hbm, xr_hbm, att_hbm, src_hbm, dst_hbm, z2_hbm, z1_hbm,
             conv_out, den_out, src_v, dst_v, xl_buf, xr_buf, wmsg_buf,
             ee_buf, att_v, conv_sh, denom_sh, sem1, sem2):
  ci = lax.axis_index("c")
  si = lax.axis_index("s")
  wid = si * NC + ci
  rows_t = NP // NS
  rbase = si * rows_t

  pltpu.sync_copy(z2_hbm, conv_sh.at[pl.ds(rbase, rows_t)])
  pltpu.sync_copy(z1_hbm, denom_sh.at[pl.ds(rbase, rows_t)])
  pltpu.sync_copy(att_hbm, att_v)
  plsc.subcore_barrier()

  ebase = wid * (EC_T * CH)
  att_vecs = [att_v[pl.ds(j * L, L)] for j in range(D_H // L)]

  def chunk(cc, carry):
    off = ebase + cc * CH
    pltpu.sync_copy(src_hbm.at[pl.ds(off, CH)], src_v)
    pltpu.sync_copy(dst_hbm.at[pl.ds(off, CH)], dst_v)
    g1 = pltpu.async_copy(xl_hbm.at[src_v], xl_buf, sem1)
    g2 = pltpu.async_copy(xr_hbm.at[dst_v], xr_buf, sem2)
    g1.wait()
    g2.wait()

    lanes = lax.iota(jnp.int32, L)

    def grp(gi, c2):
      base = gi * L
      eids = base + lanes
      accs = [jnp.zeros((L,), jnp.float32) for _ in range(4)]
      for k in range(D_H):
        kk = jnp.full((L,), k, jnp.int32)
        a = plsc.load_gather(xl_buf, [eids, kk])
        b = plsc.load_gather(xr_buf, [eids, kk])
        v = a + b
        v = jnp.maximum(v, 0.2 * v)
        accs[k % 4] = accs[k % 4] + v * att_vecs[k // L][k % L]
      ee = jnp.exp((accs[0] + accs[1]) + (accs[2] + accs[3]))
      ee_buf[pl.ds(base, L)] = ee
      for j in range(L):
        s = ee[j]
        e = base + j
        for k4 in range(D_H // L):
          wmsg_buf[e, pl.ds(k4 * L, L)] = xl_buf[e, pl.ds(k4 * L, L)] * s
      return c2

    lax.fori_loop(0, CH // L, grp, 0)

    pltpu.sync_copy(wmsg_buf, conv_sh.at[dst_v], add=True)
    pltpu.sync_copy(ee_buf, denom_sh.at[dst_v], add=True)
    return carry

  lax.fori_loop(0, EC_T, chunk, 0)

  plsc.subcore_barrier()
  pltpu.sync_copy(conv_sh.at[pl.ds(rbase, rows_t)],
                  conv_out.at[ci, pl.ds(rbase, rows_t)])
  pltpu.sync_copy(denom_sh.at[pl.ds(rbase, rows_t)],
                  den_out.at[ci, pl.ds(rbase, rows_t)])


def _post_body(c0_ref, c1_ref, d0_ref, d1_ref, h_ref, bgat_ref, rmsw_ref,
               wsw_ref, bsw_ref, vsw_ref, bv_ref, wout_ref, bout_ref, z_ref):
  den = d0_ref[...] + d1_ref[...] + 1e-16
  conv = (c0_ref[...] + c1_ref[...]) / den + bgat_ref[...]
  ms = jnp.mean(conv * conv, axis=-1, keepdims=True)
  hn = conv * lax.rsqrt(ms + 1e-6) * rmsw_ref[...]
  u = jnp.dot(hn, wsw_ref[...], preferred_element_type=jnp.float32)
  u = u + bsw_ref[...]
  g = jnp.dot(hn, vsw_ref[...], preferred_element_type=jnp.float32)
  g = g + bv_ref[...]
  act = u * jax.nn.sigmoid(u) * g
  h2 = h_ref[...] + act
  z = jnp.dot(h2, wout_ref[...], preferred_element_type=jnp.float32)
  z_ref[...] = z + bout_ref[...]


def _post(c0, c1, d0, d1, h, b_gat, rms_w, W_sw, b_sw, V_sw, b_v, W_out,
          b_out):
  grid = (N // _RB,)
  full = lambda i: (0, 0)
  return pl.pallas_call(
      _post_body,
      grid=grid,
      in_specs=[
          pl.BlockSpec((_RB, D_H), lambda i: (i, 0)),
          pl.BlockSpec((_RB, D_H), lambda i: (i, 0)),
          pl.BlockSpec((_RB, 1), lambda i: (i, 0)),
          pl.BlockSpec((_RB, 1), lambda i: (i, 0)),
          pl.BlockSpec((_RB, D_H), lambda i: (i, 0)),
          pl.BlockSpec((1, D_H), full),
          pl.BlockSpec((1, D_H), full),
          pl.BlockSpec((D_H, D_H), full),
          pl.BlockSpec((1, D_H), full),
          pl.BlockSpec((D_H, D_H), full),
          pl.BlockSpec((1, D_H), full),
          pl.BlockSpec((D_H, D_OUT), full),
          pl.BlockSpec((1, D_OUT), full),
      ],
      out_specs=pl.BlockSpec((_RB, D_OUT), lambda i: (i, 0)),
      out_shape=jax.ShapeDtypeStruct((N, D_OUT), jnp.float32),
  )(c0, c1, d0, d1, h, b_gat.reshape(1, D_H), rms_w.reshape(1, D_H), W_sw,
    b_sw.reshape(1, D_H), V_sw, b_v.reshape(1, D_H), W_out,
    b_out.reshape(1, D_OUT))


@functools.cache
def _sc_decode_kernel():
  return pl.kernel(
      _sc_decode,
      out_type=jax.ShapeDtypeStruct((E2_PAD,), jnp.float32),
      mesh=_mesh(),
      compiler_params=pltpu.CompilerParams(needs_layout_passes=False, use_tc_tiling_on_sc=False),
      scratch_types=[
          pltpu.VMEM((CH,), jnp.int32),
          pltpu.VMEM((CH,), jnp.int32),
          pltpu.VMEM((CH, D_OUT), jnp.float32),
          pltpu.VMEM((CH, D_OUT), jnp.float32),
          pltpu.VMEM((CH,), jnp.float32),
          pltpu.SemaphoreType.DMA,
          pltpu.SemaphoreType.DMA,
      ],
  )


def _sc_decode(z_hbm, aidx_hbm, bidx_hbm, out_hbm, a_v, b_v, za_buf, zb_buf,
               lg_buf, sem1, sem2):
  ci = lax.axis_index("c")
  si = lax.axis_index("s")
  wid = si * NC + ci
  ebase = wid * (DC_T * CH)

  def chunk(cc, carry):
    off = ebase + cc * CH
    pltpu.sync_copy(aidx_hbm.at[pl.ds(off, CH)], a_v)
    pltpu.sync_copy(bidx_hbm.at[pl.ds(off, CH)], b_v)
    g1 = pltpu.async_copy(z_hbm.at[a_v], za_buf, sem1)
    g2 = pltpu.async_copy(z_hbm.at[b_v], zb_buf, sem2)
    g1.wait()
    g2.wait()

    lanes = lax.iota(jnp.int32, L)

    def grp(gi, c2):
      base = gi * L
      eids = base + lanes
      accs = [jnp.zeros((L,), jnp.float32) for _ in range(4)]
      for k in range(D_OUT):
        kk = jnp.full((L,), k, jnp.int32)
        t = (plsc.load_gather(za_buf, [eids, kk]) *
             plsc.load_gather(zb_buf, [eids, kk]))
        accs[k % 4] = accs[k % 4] + t
      lg_buf[pl.ds(base, L)] = (accs[0] + accs[1]) + (accs[2] + accs[3])
      return c2

    lax.fori_loop(0, CH // L, grp, 0)
    pltpu.sync_copy(lg_buf, out_hbm.at[pl.ds(off, CH)])
    return carry

  lax.fori_loop(0, DC_T, chunk, 0)


def kernel(x, pos_edge_index, neg_edge_index, W_in, b_in, W_l, W_r, att,
           b_gat, rms_w, W_sw, b_sw, V_sw, b_v, W_out, b_out):
  h, xl, xr = _encode(x, W_in, b_in, W_l, W_r)

  src = pos_edge_index[0]
  dst = pos_edge_index[1]
  pad = E_PAD - E
  src_p = jnp.concatenate([src, jnp.zeros((pad,), jnp.int32)])
  dst_p = jnp.concatenate([dst, jnp.full((pad,), N, jnp.int32)])
  z2 = jnp.zeros((NP // NS, D_H), jnp.float32)
  z1 = jnp.zeros((NP // NS,), jnp.float32)

  conv_p, den_p = _sc_edge_kernel()(xl, xr, att, src_p, dst_p, z2, z1)

  z = _post(conv_p[0, :N], conv_p[1, :N],
            den_p[0, :N].reshape(N, 1), den_p[1, :N].reshape(N, 1), h, b_gat,
            rms_w, W_sw, b_sw, V_sw, b_v, W_out, b_out)

  pad2 = E2_PAD - E2
  a_idx = jnp.concatenate(
      [src, neg_edge_index[0], jnp.zeros((pad2,), jnp.int32)])
  b_idx = jnp.concatenate(
      [dst, neg_edge_index[1], jnp.zeros((pad2,), jnp.int32)])

  logits = _sc_decode_kernel()(z, a_idx, b_idx)
  return logits[:E2]


# speedup vs baseline: 3.2491x; 3.2491x over previous
import functools

import jax
import jax.numpy as jnp
from jax import lax
from jax.experimental import pallas as pl
from jax.experimental.pallas import tpu as pltpu
from jax.experimental.pallas import tpu_sc as plsc

N = 10000
D_IN = 128
D_H = 64
D_OUT = 32

NC = 2
NS = 16
NW = NC * NS
L = 16

NP = 10240
CH = 128
E = 320000
EC_T = 80
E_PAD = NW * EC_T * CH
E2 = 2 * E
DC_T = 158
E2_PAD = NW * DC_T * CH

@functools.cache
def _mesh():
  return plsc.VectorSubcoreMesh(core_axis_name="c", subcore_axis_name="s",
                                num_cores=NC, num_subcores=NS)


_RB = 1000


def _enc_body(x_ref, win_ref, bin_ref, wl_ref, wr_ref, h_ref, xl_ref, xr_ref):
  h = jnp.dot(x_ref[...], win_ref[...], preferred_element_type=jnp.float32)
  h = h + bin_ref[...]
  h_ref[...] = h
  xl_ref[...] = jnp.dot(h, wl_ref[...], preferred_element_type=jnp.float32)
  xr_ref[...] = jnp.dot(h, wr_ref[...], preferred_element_type=jnp.float32)


def _encode(x, W_in, b_in, W_l, W_r):
  grid = (N // _RB,)
  out = pl.pallas_call(
      _enc_body,
      grid=grid,
      in_specs=[
          pl.BlockSpec((_RB, D_IN), lambda i: (i, 0)),
          pl.BlockSpec((D_IN, D_H), lambda i: (0, 0)),
          pl.BlockSpec((1, D_H), lambda i: (0, 0)),
          pl.BlockSpec((D_H, D_H), lambda i: (0, 0)),
          pl.BlockSpec((D_H, D_H), lambda i: (0, 0)),
      ],
      out_specs=[
          pl.BlockSpec((_RB, D_H), lambda i: (i, 0)),
          pl.BlockSpec((_RB, D_H), lambda i: (i, 0)),
          pl.BlockSpec((_RB, D_H), lambda i: (i, 0)),
      ],
      out_shape=[
          jax.ShapeDtypeStruct((N, D_H), jnp.float32),
          jax.ShapeDtypeStruct((N, D_H), jnp.float32),
          jax.ShapeDtypeStruct((N, D_H), jnp.float32),
      ],
  )(x, W_in, b_in.reshape(1, D_H), W_l, W_r)
  return out


@functools.cache
def _sc_edge_kernel():
  return pl.kernel(
      _sc_edge,
      out_type=[
          jax.ShapeDtypeStruct((NC, NP, D_H), jnp.float32),
          jax.ShapeDtypeStruct((NC, NP), jnp.float32),
      ],
      mesh=_mesh(),
      compiler_params=pltpu.CompilerParams(needs_layout_passes=False, use_tc_tiling_on_sc=False),
      scratch_types=[
          pltpu.VMEM((EC_T, CH), jnp.int32),
          pltpu.VMEM((EC_T, CH), jnp.int32),
          pltpu.VMEM((CH, D_H), jnp.float32),
          pltpu.VMEM((CH, D_H), jnp.float32),
          pltpu.VMEM((CH, D_H), jnp.float32),
          pltpu.VMEM((CH, D_H), jnp.float32),
          pltpu.VMEM((CH, D_H), jnp.float32),
          pltpu.VMEM((CH,), jnp.float32),
          pltpu.VMEM((D_H,), jnp.float32),
          pltpu.VMEM_SHARED((NP, D_H), jnp.float32),
          pltpu.VMEM_SHARED((NP,), jnp.float32),
          pltpu.SemaphoreType.DMA,
          pltpu.SemaphoreType.DMA,
      ],
  )


def _sc_edge(xl_hbm, xr_hbm, att_hbm, src_hbm, dst_hbm, z2_hbm, z1_hbm,
             conv_out, den_out, src_all, dst_all, xl_a, xr_a, xl_b, xr_b,
             wmsg_buf, ee_buf, att_v, conv_sh, denom_sh, sem_a, sem_b):
  ci = lax.axis_index("c")
  si = lax.axis_index("s")
  wid = si * NC + ci
  rows_t = NP // NS
  rbase = si * rows_t

  pltpu.sync_copy(z2_hbm, conv_sh.at[pl.ds(rbase, rows_t)])
  pltpu.sync_copy(z1_hbm, denom_sh.at[pl.ds(rbase, rows_t)])
  pltpu.sync_copy(att_hbm, att_v)
  pltpu.sync_copy(src_hbm.at[pl.ds(wid * EC_T, EC_T)], src_all)
  pltpu.sync_copy(dst_hbm.at[pl.ds(wid * EC_T, EC_T)], dst_all)
  plsc.subcore_barrier()

  att_vecs = [att_v[pl.ds(j * L, L)] for j in range(D_H // L)]
  lanes = lax.iota(jnp.int32, L)

  def issue(cc, xl_t, xr_t, sem):
    pltpu.async_copy(xl_hbm.at[src_all.at[cc]], xl_t, sem)
    pltpu.async_copy(xr_hbm.at[dst_all.at[cc]], xr_t, sem)

  def wait(xl_t, xr_t, sem):
    pltpu.make_async_copy(xl_hbm.at[src_all.at[0]], xl_t, sem).wait()
    pltpu.make_async_copy(xr_hbm.at[dst_all.at[0]], xr_t, sem).wait()

  def compute(cc, xl_t, xr_t):
    def grp(gi, c2):
      base = gi * L
      acc = jnp.zeros((L,), jnp.float32)
      for j in range(L):
        e = base + j
        tot = jnp.zeros((L,), jnp.float32)
        for k4 in range(D_H // L):
          v = xl_t[e, pl.ds(k4 * L, L)] + xr_t[e, pl.ds(k4 * L, L)]
          v = jnp.maximum(v, 0.2 * v)
          tot = tot + v * att_vecs[k4]
        sc = jnp.sum(tot)
        acc = jnp.where(lanes == j, sc, acc)
      ee = jnp.exp(acc)
      ee_buf[pl.ds(base, L)] = ee
      for j in range(L):
        s = ee[j]
        e = base + j
        for k4 in range(D_H // L):
          wmsg_buf[e, pl.ds(k4 * L, L)] = xl_t[e, pl.ds(k4 * L, L)] * s
      return c2

    lax.fori_loop(0, CH // L, grp, 0)
    pltpu.sync_copy(wmsg_buf, conv_sh.at[dst_all.at[cc]], add=True)
    pltpu.sync_copy(ee_buf, denom_sh.at[dst_all.at[cc]], add=True)

  issue(0, xl_a, xr_a, sem_a)

  def pair(i, carry):
    c0 = 2 * i
    issue(c0 + 1, xl_b, xr_b, sem_b)
    wait(xl_a, xr_a, sem_a)
    compute(c0, xl_a, xr_a)

    @pl.when(c0 + 2 < EC_T)
    def _():
      issue(c0 + 2, xl_a, xr_a, sem_a)

    wait(xl_b, xr_b, sem_b)
    compute(c0 + 1, xl_b, xr_b)
    return carry

  lax.fori_loop(0, EC_T // 2, pair, 0)

  plsc.subcore_barrier()
  pltpu.sync_copy(conv_sh.at[pl.ds(rbase, rows_t)],
                  conv_out.at[ci, pl.ds(rbase, rows_t)])
  pltpu.sync_copy(denom_sh.at[pl.ds(rbase, rows_t)],
                  den_out.at[ci, pl.ds(rbase, rows_t)])


def _post_body(c0_ref, c1_ref, d0_ref, d1_ref, h_ref, bgat_ref, rmsw_ref,
               wsw_ref, bsw_ref, vsw_ref, bv_ref, wout_ref, bout_ref, z_ref):
  den = d0_ref[...] + d1_ref[...] + 1e-16
  conv = (c0_ref[...] + c1_ref[...]) / den + bgat_ref[...]
  ms = jnp.mean(conv * conv, axis=-1, keepdims=True)
  hn = conv * lax.rsqrt(ms + 1e-6) * rmsw_ref[...]
  u = jnp.dot(hn, wsw_ref[...], preferred_element_type=jnp.float32)
  u = u + bsw_ref[...]
  g = jnp.dot(hn, vsw_ref[...], preferred_element_type=jnp.float32)
  g = g + bv_ref[...]
  act = u * jax.nn.sigmoid(u) * g
  h2 = h_ref[...] + act
  z = jnp.dot(h2, wout_ref[...], preferred_element_type=jnp.float32)
  z_ref[...] = z + bout_ref[...]


def _post(c0, c1, d0, d1, h, b_gat, rms_w, W_sw, b_sw, V_sw, b_v, W_out,
          b_out):
  grid = (N // _RB,)
  full = lambda i: (0, 0)
  return pl.pallas_call(
      _post_body,
      grid=grid,
      in_specs=[
          pl.BlockSpec((_RB, D_H), lambda i: (i, 0)),
          pl.BlockSpec((_RB, D_H), lambda i: (i, 0)),
          pl.BlockSpec((_RB, 1), lambda i: (i, 0)),
          pl.BlockSpec((_RB, 1), lambda i: (i, 0)),
          pl.BlockSpec((_RB, D_H), lambda i: (i, 0)),
          pl.BlockSpec((1, D_H), full),
          pl.BlockSpec((1, D_H), full),
          pl.BlockSpec((D_H, D_H), full),
          pl.BlockSpec((1, D_H), full),
          pl.BlockSpec((D_H, D_H), full),
          pl.BlockSpec((1, D_H), full),
          pl.BlockSpec((D_H, D_OUT), full),
          pl.BlockSpec((1, D_OUT), full),
      ],
      out_specs=pl.BlockSpec((_RB, D_OUT), lambda i: (i, 0)),
      out_shape=jax.ShapeDtypeStruct((N, D_OUT), jnp.float32),
  )(c0, c1, d0, d1, h, b_gat.reshape(1, D_H), rms_w.reshape(1, D_H), W_sw,
    b_sw.reshape(1, D_H), V_sw, b_v.reshape(1, D_H), W_out,
    b_out.reshape(1, D_OUT))


@functools.cache
def _sc_decode_kernel():
  return pl.kernel(
      _sc_decode,
      out_type=jax.ShapeDtypeStruct((E2_PAD,), jnp.float32),
      mesh=_mesh(),
      compiler_params=pltpu.CompilerParams(needs_layout_passes=False, use_tc_tiling_on_sc=False),
      scratch_types=[
          pltpu.VMEM((DC_T, CH), jnp.int32),
          pltpu.VMEM((DC_T, CH), jnp.int32),
          pltpu.VMEM((CH, D_OUT), jnp.float32),
          pltpu.VMEM((CH, D_OUT), jnp.float32),
          pltpu.VMEM((CH, D_OUT), jnp.float32),
          pltpu.VMEM((CH, D_OUT), jnp.float32),
          pltpu.VMEM((DC_T * CH,), jnp.float32),
          pltpu.SemaphoreType.DMA,
          pltpu.SemaphoreType.DMA,
      ],
  )


def _sc_decode(z_hbm, aidx_hbm, bidx_hbm, out_hbm, a_all, b_all, za_a, zb_a,
               za_b, zb_b, lg_all, sem_a, sem_b):
  ci = lax.axis_index("c")
  si = lax.axis_index("s")
  wid = si * NC + ci
  ebase = wid * (DC_T * CH)

  pltpu.sync_copy(aidx_hbm.at[pl.ds(wid * DC_T, DC_T)], a_all)
  pltpu.sync_copy(bidx_hbm.at[pl.ds(wid * DC_T, DC_T)], b_all)

  lanes = lax.iota(jnp.int32, L)

  def issue(cc, za_t, zb_t, sem):
    pltpu.async_copy(z_hbm.at[a_all.at[cc]], za_t, sem)
    pltpu.async_copy(z_hbm.at[b_all.at[cc]], zb_t, sem)

  def wait(za_t, zb_t, sem):
    pltpu.make_async_copy(z_hbm.at[a_all.at[0]], za_t, sem).wait()
    pltpu.make_async_copy(z_hbm.at[b_all.at[0]], zb_t, sem).wait()

  def compute(cc, za_t, zb_t):
    def grp(gi, c2):
      base = gi * L
      acc = jnp.zeros((L,), jnp.float32)
      for j in range(L):
        e = base + j
        t = (za_t[e, pl.ds(0, L)] * zb_t[e, pl.ds(0, L)] +
             za_t[e, pl.ds(L, L)] * zb_t[e, pl.ds(L, L)])
        sc = jnp.sum(t)
        acc = jnp.where(lanes == j, sc, acc)
      lg_all[pl.ds(cc * CH + base, L)] = acc
      return c2

    lax.fori_loop(0, CH // L, grp, 0)

  issue(0, za_a, zb_a, sem_a)

  def pair(i, carry):
    c0 = 2 * i
    issue(c0 + 1, za_b, zb_b, sem_b)
    wait(za_a, zb_a, sem_a)
    compute(c0, za_a, zb_a)

    @pl.when(c0 + 2 < DC_T)
    def _():
      issue(c0 + 2, za_a, zb_a, sem_a)

    wait(za_b, zb_b, sem_b)
    compute(c0 + 1, za_b, zb_b)
    return carry

  lax.fori_loop(0, DC_T // 2, pair, 0)
  pltpu.sync_copy(lg_all, out_hbm.at[pl.ds(ebase, DC_T * CH)])


def kernel(x, pos_edge_index, neg_edge_index, W_in, b_in, W_l, W_r, att,
           b_gat, rms_w, W_sw, b_sw, V_sw, b_v, W_out, b_out):
  h, xl, xr = _encode(x, W_in, b_in, W_l, W_r)

  src = pos_edge_index[0]
  dst = pos_edge_index[1]
  pad = E_PAD - E
  src_p = jnp.concatenate([src, jnp.zeros((pad,), jnp.int32)])
  src_p = src_p.reshape(NW * EC_T, CH)
  dst_p = jnp.concatenate([dst, jnp.full((pad,), N, jnp.int32)])
  dst_p = dst_p.reshape(NW * EC_T, CH)
  z2 = jnp.zeros((NP // NS, D_H), jnp.float32)
  z1 = jnp.zeros((NP // NS,), jnp.float32)

  conv_p, den_p = _sc_edge_kernel()(xl, xr, att, src_p, dst_p, z2, z1)

  z = _post(conv_p[0, :N], conv_p[1, :N],
            den_p[0, :N].reshape(N, 1), den_p[1, :N].reshape(N, 1), h, b_gat,
            rms_w, W_sw, b_sw, V_sw, b_v, W_out, b_out)

  pad2 = E2_PAD - E2
  a_idx = jnp.concatenate(
      [src, neg_edge_index[0], jnp.zeros((pad2,), jnp.int32)])
  a_idx = a_idx.reshape(NW * DC_T, CH)
  b_idx = jnp.concatenate(
      [dst, neg_edge_index[1], jnp.zeros((pad2,), jnp.int32)])
  b_idx = b_idx.reshape(NW * DC_T, CH)

  logits = _sc_decode_kernel()(z, a_idx, b_idx)
  return logits[:E2]
